# baseline (device time: 11465 ns/iter reference)
import functools

import jax
import jax.numpy as jnp
from jax import lax
from jax.experimental import pallas as pl
from jax.experimental.pallas import tpu as pltpu

N_DEV = 4


def _coords(k: int) -> tuple[int, int]:
    return (k // 2, k % 2)


def kernel(x, dy, gamma):
    m_per, d = x.shape
    m_half = m_per // 2

    def body(x_ref, dy_ref, gamma_ref, out_ref, part_ref, comm_ref,
             send_sems, recv_sems):
        my_x = lax.axis_index("x")
        my_y = lax.axis_index("y")
        my_id = my_x * 2 + my_y

        barrier_sem = pltpu.get_barrier_semaphore()
        for k in range(N_DEV):
            @pl.when(my_id != k)
            def _():
                pl.semaphore_signal(
                    barrier_sem, inc=1,
                    device_id=_coords(k),
                    device_id_type=pl.DeviceIdType.MESH,
                )
        pl.semaphore_wait(barrier_sem, N_DEV - 1)

        row0 = my_x * m_half
        xs = x_ref[pl.ds(row0, m_half), :]
        dys = dy_ref[pl.ds(row0, m_half), :]
        mu = jnp.mean(xs, axis=1, keepdims=True)
        xc = xs - mu
        var = jnp.mean(xc * xc, axis=1, keepdims=True)
        rstd = lax.rsqrt(var + 1e-5)
        xhat = xc * rstd
        dgamma = jnp.sum(dys * xhat, axis=0)
        dbeta = jnp.sum(dys, axis=0)
        part_ref[0, :] = dgamma
        part_ref[1, :] = dbeta

        for me_k in range(N_DEV):
            @pl.when(my_id == me_k)
            def _():
                comm_ref[me_k, :, :] = part_ref[:, :]
                sends = []
                for j in range(N_DEV):
                    if j == me_k:
                        continue
                    rdma = pltpu.make_async_remote_copy(
                        src_ref=part_ref,
                        dst_ref=comm_ref.at[me_k],
                        send_sem=send_sems.at[j],
                        recv_sem=recv_sems.at[me_k],
                        device_id=_coords(j),
                        device_id_type=pl.DeviceIdType.MESH,
                    )
                    rdma.start()
                    sends.append(rdma)
                for rdma in sends:
                    rdma.wait_send()

        for j in range(N_DEV):
            @pl.when(my_id != j)
            def _():
                recv = pltpu.make_async_remote_copy(
                    src_ref=part_ref,
                    dst_ref=comm_ref.at[j],
                    send_sem=send_sems.at[j],
                    recv_sem=recv_sems.at[j],
                    device_id=_coords(j),
                    device_id_type=pl.DeviceIdType.MESH,
                )
                recv.wait_recv()

        out_ref[:, :] = (comm_ref[0] + comm_ref[1]) + (comm_ref[2] + comm_ref[3])

    return pl.pallas_call(
        body,
        out_shape=jax.ShapeDtypeStruct((2, d), jnp.float32),
        in_specs=[
            pl.BlockSpec(memory_space=pltpu.VMEM),
            pl.BlockSpec(memory_space=pltpu.VMEM),
            pl.BlockSpec(memory_space=pltpu.VMEM),
        ],
        out_specs=pl.BlockSpec(memory_space=pltpu.VMEM),
        scratch_shapes=[
            pltpu.VMEM((2, d), jnp.float32),
            pltpu.VMEM((N_DEV, 2, d), jnp.float32),
            pltpu.SemaphoreType.DMA((N_DEV,)),
            pltpu.SemaphoreType.DMA((N_DEV,)),
        ],
        compiler_params=pltpu.CompilerParams(collective_id=0),
    )(x, dy, gamma)
